# baseline (speedup 1.0000x reference)
import functools

import jax
import jax.numpy as jnp
from jax.experimental import pallas as pl
from jax.experimental.pallas import tpu as pltpu

_IMGS = 4


def _cnn_block_kernel(x_ref, w_ref, b_ref, o_ref, xp_ref, col_ref,
                      *, H, W, Cin, Cout):
    Ho, Wo = H // 2, W // 2
    M = H * W
    X0 = 128
    G = 128

    n_pairs = _IMGS // 2
    base = tuple(X0 + p * (M + G) for p in range(n_pairs))

    @pl.when(pl.program_id(0) == 0)
    def _():
        xp_ref[0:X0, :] = jnp.zeros((X0, 2 * Cin), jnp.float32)
        for p in range(n_pairs):
            xp_ref[base[p] + M:base[p] + M + G, :] = (
                jnp.zeros((G, 2 * Cin), jnp.float32))

    for p in range(n_pairs):
        for j in range(2):
            xp_ref[base[p]:base[p] + M, j * Cin:(j + 1) * Cin] = (
                x_ref[2 * p + j].reshape(M, Cin))
    xp = xp_ref[...]

    row = jax.lax.broadcasted_iota(jnp.int32, (M, 1), 0) % W
    not_first = row != 0
    not_last = row != (W - 1)

    for p in range(n_pairs):
        for dy in range(3):
            for dx in range(3):
                t = dy * 3 + dx
                s = base[p] + (dy - 1) * W + (dx - 1)
                v = xp[s:s + M, :]
                if dx == 0:
                    v = jnp.where(not_first, v, 0.0)
                elif dx == 2:
                    v = jnp.where(not_last, v, 0.0)
                vb = v.astype(jnp.bfloat16)
                col_ref[(2 * p) * M:(2 * p + 1) * M,
                        t * Cin:(t + 1) * Cin] = vb[:, 0:Cin]
                col_ref[(2 * p + 1) * M:(2 * p + 2) * M,
                        t * Cin:(t + 1) * Cin] = vb[:, Cin:2 * Cin]

    acc = jnp.dot(col_ref[...], w_ref[...], preferred_element_type=jnp.float32)

    w3 = acc.reshape(H * Wo * _IMGS, 2, Cout)
    wp = jnp.maximum(w3[:, 0, :], w3[:, 1, :])
    h4 = wp.reshape(_IMGS * Ho, 2, Wo, Cout)
    pooled = jnp.maximum(h4[:, 0], h4[:, 1])

    out = jnp.maximum(pooled + b_ref[...], 0.0)
    o_ref[...] = out.reshape(_IMGS, Ho, Wo, Cout)


def kernel(x_nchw, w_oihw, bias):
    B, Cin, H, W = x_nchw.shape
    Cout = w_oihw.shape[0]
    Ho, Wo = H // 2, W // 2
    K = 9 * Cin
    X0 = 128
    G = 128
    M = H * W

    x_nhwc = jnp.transpose(x_nchw, (0, 2, 3, 1))
    w_mat = jnp.transpose(w_oihw, (2, 3, 1, 0)).reshape(K, Cout)
    w_mat = w_mat.astype(jnp.bfloat16)
    b_row = bias.reshape(1, Cout).astype(jnp.float32)

    body = functools.partial(_cnn_block_kernel, H=H, W=W, Cin=Cin, Cout=Cout)
    out_nhwc = pl.pallas_call(
        body,
        out_shape=jax.ShapeDtypeStruct((B, Ho, Wo, Cout), x_nchw.dtype),
        grid=(B // _IMGS,),
        in_specs=[
            pl.BlockSpec((_IMGS, H, W, Cin), lambda b: (b, 0, 0, 0)),
            pl.BlockSpec((K, Cout), lambda b: (0, 0)),
            pl.BlockSpec((1, Cout), lambda b: (0, 0)),
        ],
        out_specs=pl.BlockSpec((_IMGS, Ho, Wo, Cout), lambda b: (b, 0, 0, 0)),
        scratch_shapes=[
            pltpu.VMEM((X0 + (_IMGS // 2) * (M + G), 2 * Cin), jnp.float32),
            pltpu.VMEM((_IMGS * M, K), jnp.bfloat16),
        ],
        compiler_params=pltpu.CompilerParams(
            dimension_semantics=("arbitrary",),
        ),
    )(x_nhwc, w_mat, b_row)

    return jnp.transpose(out_nhwc, (0, 3, 1, 2))
